# baseline (speedup 1.0000x reference)
import functools

import jax
import jax.numpy as jnp
from jax.experimental import pallas as pl
from jax.experimental.pallas import tpu as pltpu


def _se_ring_kernel(x_hbm, w1t_ref, b1_ref, a1_ref,
                    w2t_ref, b2_ref, a2_ref,
                    w3t_ref, b3_ref,
                    out_hbm,
                    bufs, rsems, wsems,
                    *, tb, n_groups, nbuf, depth, inv_hw):
    def read(g, s):
        pltpu.make_async_copy(
            x_hbm.at[pl.ds(g * tb, tb)], bufs.at[s], rsems.at[s]).start()

    def wait_read(s):
        pltpu.make_async_copy(
            x_hbm.at[pl.ds(0, tb)], bufs.at[s], rsems.at[s]).wait()

    def write(g, s):
        pltpu.make_async_copy(
            bufs.at[s], out_hbm.at[pl.ds(g * tb, tb)], wsems.at[s]).start()

    def wait_write(s):
        pltpu.make_async_copy(
            bufs.at[s], out_hbm.at[pl.ds(0, tb)], wsems.at[s]).wait()

    w1t = w1t_ref[...]
    w2t = w2t_ref[...]
    w3t = w3t_ref[...]
    b1 = b1_ref[...]
    b2 = b2_ref[...]
    b3 = b3_ref[...]
    a1 = a1_ref[...]
    a2 = a2_ref[...]

    def compute(s):
        x = bufs[s]
        gap = jnp.sum(x, axis=1) * inv_hw
        y = jnp.dot(gap, w1t, preferred_element_type=jnp.float32) + b1
        y = jnp.where(y >= 0.0, y, a1 * y)
        y = jnp.dot(y, w2t, preferred_element_type=jnp.float32) + b2
        y = jnp.where(y >= 0.0, y, a2 * y)
        y = jnp.dot(y, w3t, preferred_element_type=jnp.float32) + b3
        gate = jax.nn.sigmoid(y).astype(x.dtype)
        bufs[s] = x * gate[:, None, :]

    pending_write = [False] * nbuf

    for g in range(min(depth, n_groups)):
        read(g, g % nbuf)
    for g in range(n_groups):
        s = g % nbuf
        wait_read(s)
        compute(s)
        write(g, s)
        pending_write[s] = True
        nxt = g + depth
        if nxt < n_groups:
            s2 = nxt % nbuf
            if pending_write[s2]:
                wait_write(s2)
                pending_write[s2] = False
            read(nxt, s2)
    for s in range(nbuf):
        if pending_write[s]:
            wait_write(s)


@jax.jit
def kernel(x, w1, b1, a1, w2, b2, a2, w3, b3):
    B, C, H, W = x.shape
    hw = H * W

    xt = jnp.transpose(x, (0, 2, 3, 1)).reshape(B, hw, C)

    tb = 4
    while B % tb:
        tb //= 2
    n_groups = B // tb
    nbuf = min(6, n_groups)
    depth = min(4, nbuf - 1) if nbuf > 1 else 1

    w1t = w1.T
    w2t = w2.T
    w3t = w3.T

    vmem = lambda shape: pl.BlockSpec(shape, lambda: tuple(0 for _ in shape))
    any_spec = pl.BlockSpec(memory_space=pl.ANY)

    buf_bytes = nbuf * tb * hw * C * 4
    weight_bytes = 3 * C * C * 4 + 5 * C * 4
    vmem_limit = int(min(100 * 2**20, buf_bytes + 2 * weight_bytes + 2**20))

    body = functools.partial(
        _se_ring_kernel,
        tb=tb, n_groups=n_groups, nbuf=nbuf, depth=depth,
        inv_hw=1.0 / float(hw))

    outt = pl.pallas_call(
        body,
        out_shape=jax.ShapeDtypeStruct((B, hw, C), x.dtype),
        in_specs=[
            any_spec,
            vmem((C, C)), vmem((1, C)), vmem((1, C)),
            vmem((C, C)), vmem((1, C)), vmem((1, C)),
            vmem((C, C)), vmem((1, C)),
        ],
        out_specs=any_spec,
        scratch_shapes=[
            pltpu.VMEM((nbuf, tb, hw, C), jnp.float32),
            pltpu.SemaphoreType.DMA((nbuf,)),
            pltpu.SemaphoreType.DMA((nbuf,)),
        ],
        compiler_params=pltpu.CompilerParams(
            vmem_limit_bytes=vmem_limit,
        ),
    )(
        xt,
        w1t, b1, a1,
        w2t, b2, a2,
        w3t, b3,
    )
    return outt.reshape(B, H, W, C).transpose(0, 3, 1, 2)


# speedup vs baseline: 1.0277x; 1.0277x over previous
import functools

import jax
import jax.numpy as jnp
from jax.experimental import pallas as pl
from jax.experimental.pallas import tpu as pltpu


def _se_ring_kernel(x_hbm, w1t_ref, b1_ref, a1_ref,
                    w2t_ref, b2_ref, a2_ref,
                    w3t_ref, b3_ref,
                    out_hbm,
                    bufs, rsems, wsems,
                    *, tb, n_groups, nbuf, depth, inv_hw):
    def read(g, s):
        pltpu.make_async_copy(
            x_hbm.at[pl.ds(g * tb, tb)], bufs.at[s], rsems.at[s]).start()

    def wait_read(s):
        pltpu.make_async_copy(
            x_hbm.at[pl.ds(0, tb)], bufs.at[s], rsems.at[s]).wait()

    def write(g, s):
        pltpu.make_async_copy(
            bufs.at[s], out_hbm.at[pl.ds(g * tb, tb)], wsems.at[s]).start()

    def wait_write(s):
        pltpu.make_async_copy(
            bufs.at[s], out_hbm.at[pl.ds(0, tb)], wsems.at[s]).wait()

    w1t = w1t_ref[...]
    w2t = w2t_ref[...]
    w3t = w3t_ref[...]
    b1 = b1_ref[...]
    b2 = b2_ref[...]
    b3 = b3_ref[...]
    a1 = a1_ref[...]
    a2 = a2_ref[...]

    def compute(s):
        x = bufs[s]
        gap = jnp.sum(x, axis=1) * inv_hw
        y = jnp.dot(gap, w1t, preferred_element_type=jnp.float32) + b1
        y = jnp.where(y >= 0.0, y, a1 * y)
        y = jnp.dot(y, w2t, preferred_element_type=jnp.float32) + b2
        y = jnp.where(y >= 0.0, y, a2 * y)
        y = jnp.dot(y, w3t, preferred_element_type=jnp.float32) + b3
        gate = jax.nn.sigmoid(y).astype(x.dtype)
        bufs[s] = x * gate[:, None, :]

    pending_write = [False] * nbuf

    for g in range(min(depth, n_groups)):
        read(g, g % nbuf)
    for g in range(n_groups):
        s = g % nbuf
        wait_read(s)
        compute(s)
        write(g, s)
        pending_write[s] = True
        nxt = g + depth
        if nxt < n_groups:
            s2 = nxt % nbuf
            if pending_write[s2]:
                wait_write(s2)
                pending_write[s2] = False
            read(nxt, s2)
    for s in range(nbuf):
        if pending_write[s]:
            wait_write(s)


@jax.jit
def kernel(x, w1, b1, a1, w2, b2, a2, w3, b3):
    B, C, H, W = x.shape
    hw = H * W

    xt = jnp.transpose(x, (0, 2, 3, 1)).reshape(B, hw, C)

    tb = 2
    while B % tb:
        tb //= 2
    n_groups = B // tb
    nbuf = min(10, n_groups)
    depth = min(6, nbuf - 1) if nbuf > 1 else 1

    w1t = w1.T
    w2t = w2.T
    w3t = w3.T

    vmem = lambda shape: pl.BlockSpec(shape, lambda: tuple(0 for _ in shape))
    any_spec = pl.BlockSpec(memory_space=pl.ANY)

    buf_bytes = nbuf * tb * hw * C * 4
    weight_bytes = 3 * C * C * 4 + 5 * C * 4
    vmem_limit = int(min(100 * 2**20, buf_bytes + 2 * weight_bytes + 2**20))

    body = functools.partial(
        _se_ring_kernel,
        tb=tb, n_groups=n_groups, nbuf=nbuf, depth=depth,
        inv_hw=1.0 / float(hw))

    outt = pl.pallas_call(
        body,
        out_shape=jax.ShapeDtypeStruct((B, hw, C), x.dtype),
        in_specs=[
            any_spec,
            vmem((C, C)), vmem((1, C)), vmem((1, C)),
            vmem((C, C)), vmem((1, C)), vmem((1, C)),
            vmem((C, C)), vmem((1, C)),
        ],
        out_specs=any_spec,
        scratch_shapes=[
            pltpu.VMEM((nbuf, tb, hw, C), jnp.float32),
            pltpu.SemaphoreType.DMA((nbuf,)),
            pltpu.SemaphoreType.DMA((nbuf,)),
        ],
        compiler_params=pltpu.CompilerParams(
            vmem_limit_bytes=vmem_limit,
        ),
    )(
        xt,
        w1t, b1, a1,
        w2t, b2, a2,
        w3t, b3,
    )
    return outt.reshape(B, H, W, C).transpose(0, 3, 1, 2)


# speedup vs baseline: 1.0359x; 1.0080x over previous
import functools

import jax
import jax.numpy as jnp
from jax.experimental import pallas as pl
from jax.experimental.pallas import tpu as pltpu


def _se_ring_kernel(x_hbm, w1t_ref, b1_ref, a1_ref,
                    w2t_ref, b2_ref, a2_ref,
                    w3t_ref, b3_ref,
                    out_hbm,
                    bufs, rsems, wsems,
                    *, tb, n_groups, nbuf, depth, inv_hw):
    def read(g, s):
        pltpu.make_async_copy(
            x_hbm.at[pl.ds(g * tb, tb)], bufs.at[s], rsems.at[s]).start()

    def wait_read(s):
        pltpu.make_async_copy(
            x_hbm.at[pl.ds(0, tb)], bufs.at[s], rsems.at[s]).wait()

    def write(g, s):
        pltpu.make_async_copy(
            bufs.at[s], out_hbm.at[pl.ds(g * tb, tb)], wsems.at[s]).start()

    def wait_write(s):
        pltpu.make_async_copy(
            bufs.at[s], out_hbm.at[pl.ds(0, tb)], wsems.at[s]).wait()

    w1t = w1t_ref[...]
    w2t = w2t_ref[...]
    w3t = w3t_ref[...]
    b1 = b1_ref[...]
    b2 = b2_ref[...]
    b3 = b3_ref[...]
    a1 = a1_ref[...]
    a2 = a2_ref[...]

    def compute(s):
        x = bufs[s]
        gap = jnp.sum(x, axis=1) * inv_hw
        y = jnp.dot(gap, w1t, preferred_element_type=jnp.float32) + b1
        y = jnp.where(y >= 0.0, y, a1 * y)
        y = jnp.dot(y, w2t, preferred_element_type=jnp.float32) + b2
        y = jnp.where(y >= 0.0, y, a2 * y)
        y = jnp.dot(y, w3t, preferred_element_type=jnp.float32) + b3
        gate = jax.nn.sigmoid(y).astype(x.dtype)
        bufs[s] = x * gate[:, None, :]

    pending_write = [False] * nbuf

    for g in range(min(depth, n_groups)):
        read(g, g % nbuf)
    for g in range(n_groups):
        s = g % nbuf
        wait_read(s)
        compute(s)
        write(g, s)
        pending_write[s] = True
        nxt = g + depth
        if nxt < n_groups:
            s2 = nxt % nbuf
            if pending_write[s2]:
                wait_write(s2)
                pending_write[s2] = False
            read(nxt, s2)
    for s in range(nbuf):
        if pending_write[s]:
            wait_write(s)


@jax.jit
def kernel(x, w1, b1, a1, w2, b2, a2, w3, b3):
    B, C, H, W = x.shape
    hw = H * W

    xt = jnp.transpose(x, (0, 2, 3, 1)).reshape(B, hw, C)

    tb = 4
    while B % tb:
        tb //= 2
    n_groups = B // tb
    nbuf = min(5, n_groups)
    depth = min(3, nbuf - 1) if nbuf > 1 else 1

    w1t = w1.T
    w2t = w2.T
    w3t = w3.T

    vmem = lambda shape: pl.BlockSpec(shape, lambda: tuple(0 for _ in shape))
    any_spec = pl.BlockSpec(memory_space=pl.ANY)

    buf_bytes = nbuf * tb * hw * C * 4
    weight_bytes = 3 * C * C * 4 + 5 * C * 4
    vmem_limit = int(min(100 * 2**20, buf_bytes + 2 * weight_bytes + 2**20))

    body = functools.partial(
        _se_ring_kernel,
        tb=tb, n_groups=n_groups, nbuf=nbuf, depth=depth,
        inv_hw=1.0 / float(hw))

    outt = pl.pallas_call(
        body,
        out_shape=jax.ShapeDtypeStruct((B, hw, C), x.dtype),
        in_specs=[
            any_spec,
            vmem((C, C)), vmem((1, C)), vmem((1, C)),
            vmem((C, C)), vmem((1, C)), vmem((1, C)),
            vmem((C, C)), vmem((1, C)),
        ],
        out_specs=any_spec,
        scratch_shapes=[
            pltpu.VMEM((nbuf, tb, hw, C), jnp.float32),
            pltpu.SemaphoreType.DMA((nbuf,)),
            pltpu.SemaphoreType.DMA((nbuf,)),
        ],
        compiler_params=pltpu.CompilerParams(
            vmem_limit_bytes=vmem_limit,
        ),
    )(
        xt,
        w1t, b1, a1,
        w2t, b2, a2,
        w3t, b3,
    )
    return outt.reshape(B, H, W, C).transpose(0, 3, 1, 2)
